# baseline (speedup 1.0000x reference)
import functools

import jax
import jax.numpy as jnp
from jax.experimental import pallas as pl
from jax.experimental.pallas import tpu as pltpu

D = 1024
DH = 1024
E = 8
F = 2048
R = 16
ER = E * R

S_BLK = 256
LN_EPS = 1e-5


def _moe_kernel(x_ref, hist_ref, persona_ref, W1_ref, b1_ref, W2_ref, b2_ref,
                lng_ref, lnb_ref, gw_ref, gb_ref, Wg_ref, Wu_ref, Wd_ref,
                AgF_ref, BgF_ref, AuF_ref, BuF_ref, AdF_ref, BdF_ref,
                out_ref):
    xb = x_ref[...]
    W1 = W1_ref[...]
    W1x = W1[:, :D]
    W1h = W1[:, D:D + DH]
    W1p = W1[:, D + DH:]

    base1 = (jnp.dot(xb, W1x.T) + jnp.dot(hist_ref[...], W1h.T)
             + b1_ref[...])
    pc = jnp.dot(persona_ref[...], W1p.T)
    W2 = W2_ref[...]
    b2 = b2_ref[...]
    gw = lng_ref[...] * gw_ref[...]
    gw_sum = jnp.sum(gw)
    cterm = jnp.sum(lnb_ref[...] * gw_ref[...]) + gb_ref[0, 0]

    cols = []
    for e in range(E):
        h1 = jax.nn.relu(base1 + pc[e][None, :])
        h2 = jax.nn.relu(jnp.dot(h1, W2.T) + b2)
        m = jnp.mean(h2, axis=1, keepdims=True)
        v = jnp.mean((h2 - m) ** 2, axis=1, keepdims=True)
        rstd = jax.lax.rsqrt(v + LN_EPS)
        lg = (jnp.dot(h2, gw.T) - m * gw_sum) * rstd + cterm
        cols.append(lg)
    logits = jnp.concatenate(cols, axis=1)

    mx = jnp.max(logits, axis=1, keepdims=True)
    ex = jnp.exp(logits - mx)
    p = ex / jnp.sum(ex, axis=1, keepdims=True)
    iota_e = jax.lax.broadcasted_iota(jnp.int32, (S_BLK, E), 1)
    w1 = jnp.max(p, axis=1, keepdims=True)
    idx1 = jnp.min(jnp.where(p >= w1, iota_e, E), axis=1, keepdims=True)
    oh1 = (iota_e == idx1).astype(jnp.float32)
    p2 = jnp.where(iota_e == idx1, -1.0, p)
    w2 = jnp.max(p2, axis=1, keepdims=True)
    idx2 = jnp.min(jnp.where(p2 >= w2, iota_e, E), axis=1, keepdims=True)
    oh2 = (iota_e == idx2).astype(jnp.float32)

    f32 = jnp.float32
    xb16 = xb.astype(jnp.bfloat16)
    G = jnp.dot(xb16, Wg_ref[...].T, preferred_element_type=f32)
    U = jnp.dot(xb16, Wu_ref[...].T, preferred_element_type=f32)
    zg = jnp.dot(xb16, AgF_ref[...].T, preferred_element_type=f32)
    zu = jnp.dot(xb16, AuF_ref[...].T, preferred_element_type=f32)

    exp_e = jax.lax.broadcasted_iota(jnp.int32, (E, ER), 0)
    exp_c = jax.lax.broadcasted_iota(jnp.int32, (E, ER), 1)
    expand = (exp_c // R == exp_e).astype(jnp.float32)

    acc_a = jnp.zeros((S_BLK, F), jnp.float32)
    acc_y = jnp.zeros((S_BLK, ER), jnp.float32)
    for oh, w in ((oh1, w1), (oh2, w2)):
        mask = jnp.dot(oh, expand)
        g = G + jnp.dot((zg * mask).astype(jnp.bfloat16), BgF_ref[...].T,
                        preferred_element_type=f32)
        u = U + jnp.dot((zu * mask).astype(jnp.bfloat16), BuF_ref[...].T,
                        preferred_element_type=f32)
        a = g * jax.lax.logistic(g) * u
        acc_a = acc_a + w * a
        ya = jnp.dot(a.astype(jnp.bfloat16), AdF_ref[...].T,
                     preferred_element_type=f32)
        acc_y = acc_y + (w * ya) * mask

    out_ref[...] = (jnp.dot(acc_a.astype(jnp.bfloat16), Wd_ref[...].T,
                            preferred_element_type=f32)
                    + jnp.dot(acc_y.astype(jnp.bfloat16), BdF_ref[...].T,
                              preferred_element_type=f32))


def kernel(x, history_hidden_embedding, persona_embedding, W1, b1, W2, b2,
           ln_g, ln_b, gate_w, gate_b, Wg, Wu, Wd, Ag, Bg, Au, Bu, Ad, Bd):
    B, S, _ = x.shape
    xf = x.reshape(B * S, D)
    n_blk = (B * S) // S_BLK

    bf16 = jnp.bfloat16
    AgF = Ag.reshape(ER, D).astype(bf16)
    AuF = Au.reshape(ER, D).astype(bf16)
    AdF = Ad.reshape(ER, F).astype(bf16)
    BgF = jnp.transpose(Bg, (1, 0, 2)).reshape(F, ER).astype(bf16)
    BuF = jnp.transpose(Bu, (1, 0, 2)).reshape(F, ER).astype(bf16)
    BdF = jnp.transpose(Bd, (1, 0, 2)).reshape(D, ER).astype(bf16)
    Wg16 = Wg.astype(bf16)
    Wu16 = Wu.astype(bf16)
    Wd16 = Wd.astype(bf16)

    inv = lambda shape: pl.BlockSpec(shape, lambda i: (0,) * len(shape))

    out = pl.pallas_call(
        _moe_kernel,
        grid=(n_blk,),
        in_specs=[
            pl.BlockSpec((S_BLK, D), lambda i: (i, 0)),
            inv((1, DH)),
            inv((E, D)),
            inv((128, D + DH + D)),
            inv((1, 128)),
            inv((D, 128)),
            inv((1, D)),
            inv((1, D)),
            inv((1, D)),
            inv((1, D)),
            inv((1, 1)),
            inv((F, D)),
            inv((F, D)),
            inv((D, F)),
            inv((ER, D)),
            inv((F, ER)),
            inv((ER, D)),
            inv((F, ER)),
            inv((ER, F)),
            inv((D, ER)),
        ],
        out_specs=pl.BlockSpec((S_BLK, D), lambda i: (i, 0)),
        out_shape=jax.ShapeDtypeStruct((B * S, D), jnp.float32),
    )(xf, history_hidden_embedding, persona_embedding, W1,
      b1.reshape(1, 128), W2, b2.reshape(1, D), ln_g.reshape(1, D),
      ln_b.reshape(1, D), gate_w, gate_b.reshape(1, 1),
      Wg16, Wu16, Wd16, AgF, BgF, AuF, BuF, AdF, BdF)

    return out.reshape(B, S, D)


# speedup vs baseline: 1.1615x; 1.1615x over previous
import functools

import jax
import jax.numpy as jnp
from jax.experimental import pallas as pl
from jax.experimental.pallas import tpu as pltpu

D = 1024
DH = 1024
E = 8
F = 2048
R = 16
ER = E * R

S_BLK = 512
LN_EPS = 1e-5


def _moe_kernel(x_ref, hist_ref, persona_ref, W1_ref, b1_ref, W2_ref, b2_ref,
                lng_ref, lnb_ref, gw_ref, gb_ref, Wg_ref, Wu_ref, Wd_ref,
                AgF_ref, BgF_ref, AuF_ref, BuF_ref, AdF_ref, BdF_ref,
                out_ref):
    xb = x_ref[...]
    W1 = W1_ref[...]
    W1x = W1[:, :D]
    W1h = W1[:, D:D + DH]
    W1p = W1[:, D + DH:]

    base1 = (jnp.dot(xb, W1x.T) + jnp.dot(hist_ref[...], W1h.T)
             + b1_ref[...])
    pc = jnp.dot(persona_ref[...], W1p.T)
    W2 = W2_ref[...]
    b2 = b2_ref[...]
    gw = lng_ref[...] * gw_ref[...]
    gw_sum = jnp.sum(gw)
    cterm = jnp.sum(lnb_ref[...] * gw_ref[...]) + gb_ref[0, 0]

    cols = []
    for e in range(E):
        h1 = jax.nn.relu(base1 + pc[e][None, :])
        h2 = jax.nn.relu(jnp.dot(h1, W2.T) + b2)
        m = jnp.mean(h2, axis=1, keepdims=True)
        v = jnp.mean((h2 - m) ** 2, axis=1, keepdims=True)
        rstd = jax.lax.rsqrt(v + LN_EPS)
        lg = (jnp.dot(h2, gw.T) - m * gw_sum) * rstd + cterm
        cols.append(lg)
    logits = jnp.concatenate(cols, axis=1)

    mx = jnp.max(logits, axis=1, keepdims=True)
    ex = jnp.exp(logits - mx)
    p = ex / jnp.sum(ex, axis=1, keepdims=True)
    iota_e = jax.lax.broadcasted_iota(jnp.int32, (S_BLK, E), 1)
    w1 = jnp.max(p, axis=1, keepdims=True)
    idx1 = jnp.min(jnp.where(p >= w1, iota_e, E), axis=1, keepdims=True)
    oh1 = (iota_e == idx1).astype(jnp.float32)
    p2 = jnp.where(iota_e == idx1, -1.0, p)
    w2 = jnp.max(p2, axis=1, keepdims=True)
    idx2 = jnp.min(jnp.where(p2 >= w2, iota_e, E), axis=1, keepdims=True)
    oh2 = (iota_e == idx2).astype(jnp.float32)

    G = jnp.dot(xb, Wg_ref[...].T)
    U = jnp.dot(xb, Wu_ref[...].T)
    zg = jnp.dot(xb, AgF_ref[...].T)
    zu = jnp.dot(xb, AuF_ref[...].T)

    exp_e = jax.lax.broadcasted_iota(jnp.int32, (E, ER), 0)
    exp_c = jax.lax.broadcasted_iota(jnp.int32, (E, ER), 1)
    expand = (exp_c // R == exp_e).astype(jnp.float32)

    acc_a = jnp.zeros((S_BLK, F), jnp.float32)
    acc_y = jnp.zeros((S_BLK, ER), jnp.float32)
    for oh, w in ((oh1, w1), (oh2, w2)):
        mask = jnp.dot(oh, expand)
        g = G + jnp.dot(zg * mask, BgF_ref[...].T)
        u = U + jnp.dot(zu * mask, BuF_ref[...].T)
        a = g * jax.lax.logistic(g) * u
        acc_a = acc_a + w * a
        ya = jnp.dot(a, AdF_ref[...].T)
        acc_y = acc_y + (w * ya) * mask

    out_ref[...] = (jnp.dot(acc_a, Wd_ref[...].T)
                    + jnp.dot(acc_y, BdF_ref[...].T))


def kernel(x, history_hidden_embedding, persona_embedding, W1, b1, W2, b2,
           ln_g, ln_b, gate_w, gate_b, Wg, Wu, Wd, Ag, Bg, Au, Bu, Ad, Bd):
    B, S, _ = x.shape
    xf = x.reshape(B * S, D)
    n_blk = (B * S) // S_BLK

    AgF = Ag.reshape(ER, D)
    AuF = Au.reshape(ER, D)
    AdF = Ad.reshape(ER, F)
    BgF = jnp.transpose(Bg, (1, 0, 2)).reshape(F, ER)
    BuF = jnp.transpose(Bu, (1, 0, 2)).reshape(F, ER)
    BdF = jnp.transpose(Bd, (1, 0, 2)).reshape(D, ER)

    inv = lambda shape: pl.BlockSpec(shape, lambda i: (0,) * len(shape))

    out = pl.pallas_call(
        _moe_kernel,
        grid=(n_blk,),
        in_specs=[
            pl.BlockSpec((S_BLK, D), lambda i: (i, 0)),
            inv((1, DH)),
            inv((E, D)),
            inv((128, D + DH + D)),
            inv((1, 128)),
            inv((D, 128)),
            inv((1, D)),
            inv((1, D)),
            inv((1, D)),
            inv((1, D)),
            inv((1, 1)),
            inv((F, D)),
            inv((F, D)),
            inv((D, F)),
            inv((ER, D)),
            inv((F, ER)),
            inv((ER, D)),
            inv((F, ER)),
            inv((ER, F)),
            inv((D, ER)),
        ],
        out_specs=pl.BlockSpec((S_BLK, D), lambda i: (i, 0)),
        out_shape=jax.ShapeDtypeStruct((B * S, D), jnp.float32),
    )(xf, history_hidden_embedding, persona_embedding, W1,
      b1.reshape(1, 128), W2, b2.reshape(1, D), ln_g.reshape(1, D),
      ln_b.reshape(1, D), gate_w, gate_b.reshape(1, 1),
      Wg, Wu, Wd, AgF, BgF, AuF, BuF, AdF, BdF)

    return out.reshape(B, S, D)
